# Initial kernel scaffold; baseline (speedup 1.0000x reference)
import jax
import jax.numpy as jnp
from jax.experimental import pallas as pl


def kernel(features, edge_index):
    raise NotImplementedError("write your pallas kernel here")


# speedup vs baseline: 3.3261x; 3.3261x over previous
import functools

import jax
import jax.numpy as jnp
from jax import lax
from jax.experimental import pallas as pl
from jax.experimental.pallas import tpu as pltpu
from jax.experimental.pallas import tpu_sc as plsc

N = 10000
E = 320000
D = 128
NC = 2
NS = 16
NW = NC * NS
CHUNK = 128
NPAD = 10240
RPT = NPAD // NS
EPAD = 327680
CPW = EPAD // (NW * CHUNK)

_F32 = jnp.float32
_MESH = dict(core_axis_name="c", subcore_axis_name="s")


def _sc_layer(table, src3, dst3, with_deg):
    half = CPW // 2
    out_type = [jax.ShapeDtypeStruct((NPAD, D), _F32)] * 2
    scratch = [
        pltpu.VMEM((half, CHUNK), jnp.int32),
        pltpu.VMEM((half, CHUNK), jnp.int32),
        pltpu.VMEM((CHUNK, D), _F32),
        pltpu.VMEM((CHUNK, D), _F32),
        pltpu.VMEM_SHARED((NPAD, D), _F32),
        pltpu.SemaphoreType.DMA,
        pltpu.SemaphoreType.DMA,
    ]
    if with_deg:
        out_type += [jax.ShapeDtypeStruct((NPAD,), _F32)] * 2
        scratch += [
            pltpu.VMEM((CHUNK,), _F32),
            pltpu.VMEM((RPT,), _F32),
            pltpu.VMEM_SHARED((NPAD,), _F32),
        ]

    def body(table_h, src_h, dst_h, *rest):
        if with_deg:
            out0, out1, dga, dgb = rest[:4]
            (src_v, dst_v, rows0, rows1, acc, sem0, sem1,
             ones_v, dz_v, dacc) = rest[4:]
        else:
            out0, out1 = rest[:2]
            src_v, dst_v, rows0, rows1, acc, sem0, sem1 = rest[2:]
        c = lax.axis_index("c")
        s = lax.axis_index("s")
        wid = c * NS + s

        def _zrow(i, carry):
            for j in range(D // 16):
                rows0[i, pl.ds(j * 16, 16)] = jnp.zeros((16,), _F32)
            return carry
        lax.fori_loop(0, CHUNK, _zrow, 0)
        for k in range(RPT // CHUNK):
            pltpu.sync_copy(rows0, acc.at[pl.ds(s * RPT + k * CHUNK, CHUNK)])
        if with_deg:
            for j in range(CHUNK // 16):
                ones_v[pl.ds(j * 16, 16)] = jnp.ones((16,), _F32)
            for j in range(RPT // 16):
                dz_v[pl.ds(j * 16, 16)] = jnp.zeros((16,), _F32)
            pltpu.sync_copy(dz_v, dacc.at[pl.ds(s * RPT, RPT)])
        plsc.subcore_barrier()

        def _chunk(i, carry):
            j0 = 2 * i
            cp0 = pltpu.async_copy(table_h.at[src_v.at[j0]], rows0, sem0)
            cp1 = pltpu.async_copy(table_h.at[src_v.at[j0 + 1]], rows1, sem1)
            cp0.wait()
            pltpu.sync_copy(rows0, acc.at[dst_v.at[j0]], add=True)
            if with_deg:
                pltpu.sync_copy(ones_v, dacc.at[dst_v.at[j0]], add=True)
            cp1.wait()
            pltpu.sync_copy(rows1, acc.at[dst_v.at[j0 + 1]], add=True)
            if with_deg:
                pltpu.sync_copy(ones_v, dacc.at[dst_v.at[j0 + 1]], add=True)
            return carry

        for h in range(2):
            pltpu.sync_copy(src_h.at[wid, pl.ds(h * half, half)], src_v)
            pltpu.sync_copy(dst_h.at[wid, pl.ds(h * half, half)], dst_v)
            lax.fori_loop(0, half // 2, _chunk, 0)
        plsc.subcore_barrier()

        row0 = s * RPT
        sl = pl.ds(row0, RPT)

        @pl.when(c == 0)
        def _():
            pltpu.sync_copy(acc.at[sl], out0.at[sl])
            if with_deg:
                pltpu.sync_copy(dacc.at[sl], dga.at[sl])

        @pl.when(c == 1)
        def _():
            pltpu.sync_copy(acc.at[sl], out1.at[sl])
            if with_deg:
                pltpu.sync_copy(dacc.at[sl], dgb.at[sl])

    f = pl.kernel(
        body,
        out_type=out_type,
        mesh=plsc.VectorSubcoreMesh(**_MESH),
        scratch_types=scratch,
    )
    return f(table, src3, dst3)


def _tc_combine(p0, p1, d0, d1):
    br = 512

    def body(p0_ref, p1_ref, d0_ref, d1_ref, x_ref, inv_ref):
        deg = d0_ref[...] + d1_ref[...]
        inv = 1.0 / jnp.maximum(deg, 1.0)
        inv_ref[...] = inv
        x_ref[...] = (p0_ref[...] + p1_ref[...]) * inv

    return pl.pallas_call(
        body,
        grid=(NPAD // br,),
        in_specs=[
            pl.BlockSpec((br, D), lambda i: (i, 0)),
            pl.BlockSpec((br, D), lambda i: (i, 0)),
            pl.BlockSpec((br, 1), lambda i: (i, 0)),
            pl.BlockSpec((br, 1), lambda i: (i, 0)),
        ],
        out_specs=[
            pl.BlockSpec((br, D), lambda i: (i, 0)),
            pl.BlockSpec((br, 1), lambda i: (i, 0)),
        ],
        out_shape=[
            jax.ShapeDtypeStruct((NPAD, D), _F32),
            jax.ShapeDtypeStruct((NPAD, 1), _F32),
        ],
    )(p0, p1, d0, d1)


def _tc_final(q0, q1, inv, x):
    br = 400

    def body(q0_ref, q1_ref, inv_ref, x_ref, o_ref):
        x2 = (q0_ref[...] + q1_ref[...]) * inv_ref[...]
        o_ref[...] = 0.5 * (x_ref[...] + x2)

    return pl.pallas_call(
        body,
        grid=(N // br,),
        in_specs=[
            pl.BlockSpec((br, D), lambda i: (i, 0)),
            pl.BlockSpec((br, D), lambda i: (i, 0)),
            pl.BlockSpec((br, 1), lambda i: (i, 0)),
            pl.BlockSpec((br, D), lambda i: (i, 0)),
        ],
        out_specs=pl.BlockSpec((br, D), lambda i: (i, 0)),
        out_shape=jax.ShapeDtypeStruct((N, D), _F32),
    )(q0, q1, inv, x)


def kernel(features, edge_index):
    src = edge_index[0]
    dst = edge_index[1]
    pad = EPAD - E
    srcp = jnp.concatenate([src, jnp.zeros((pad,), jnp.int32)])
    dstp = jnp.concatenate([dst, jnp.full((pad,), NPAD - 1, jnp.int32)])
    src3 = srcp.reshape(NW, CPW, CHUNK)
    dst3 = dstp.reshape(NW, CPW, CHUNK)

    p0, p1, dg0, dg1 = _sc_layer(features, src3, dst3, with_deg=True)
    x, inv = _tc_combine(p0, p1, dg0.reshape(NPAD, 1), dg1.reshape(NPAD, 1))
    q0, q1 = _sc_layer(x, src3, dst3, with_deg=False)
    return _tc_final(q0, q1, inv, x)
